# baseline (speedup 1.0000x reference)
import functools

import jax
import jax.numpy as jnp
from jax import lax
from jax.experimental import pallas as pl
from jax.experimental.pallas import tpu as pltpu
from jax.experimental.pallas import tpu_sc as plsc

B, V, D = 4, 32, 128
N = V * V * V
NC, NS = 2, 16
NW = NC * NS
BN = B * N
BN_W = BN // NW
ROWS_W = BN_W * 3
GR = 128
GROUPS = ROWS_W // GR
RING = 6
PF = 4
WAVES = GROUPS // RING


def _sc_body(hv_ref, i1_ref, i2_ref, i3_ref, out_ref,
             idx1_v, idx2_v, idx3_v, comb_v, rows_v, table_sh, *sems):
    gsems = sems[:RING]
    ssems = sems[RING:]

    wid = lax.axis_index("s") * NC + lax.axis_index("c")
    b = wid // (N // BN_W)
    n0 = (wid % (N // BN_W)) * BN_W
    b_off = b * V
    row0 = wid * ROWS_W

    @pl.when(lax.axis_index("s") == 0)
    def _():
        pltpu.sync_copy(hv_ref, rows_v.at[0])
        pltpu.sync_copy(rows_v.at[0], table_sh)
    plsc.subcore_barrier()

    pltpu.sync_copy(i1_ref.at[pl.ds(n0, BN_W)], idx1_v)
    pltpu.sync_copy(i2_ref.at[pl.ds(n0, BN_W)], idx2_v)
    pltpu.sync_copy(i3_ref.at[pl.ds(n0, BN_W)], idx3_v)

    lane = lax.iota(jnp.int32, 16)

    def fill(c, carry):
        base = c * 16
        for si, src in enumerate((idx1_v, idx2_v, idx3_v)):
            vals = src[pl.ds(base, 16)] + b_off
            p = (base + lane) * 3 + si
            prow = lax.shift_right_logical(p, 7)
            pcol = lax.bitwise_and(p, 127)
            plsc.store_scatter(comb_v, [prow, pcol], vals)
        return carry
    lax.fori_loop(0, BN_W // 16, fill, 0)

    def _src(r):
        return table_sh

    def start_gather(g, r):
        pltpu.async_copy(_src(r).at[comb_v.at[g]], rows_v.at[r], gsems[r])

    def wait_gather(g, r):
        pltpu.make_async_copy(
            _src(r).at[comb_v.at[g]], rows_v.at[r], gsems[r]).wait()

    def start_scatter(g, r):
        pltpu.async_copy(
            rows_v.at[r], out_ref.at[pl.ds(row0 + g * GR, GR)], ssems[r])

    def wait_scatter(r):
        pltpu.make_async_copy(
            rows_v.at[r], out_ref.at[pl.ds(row0, GR)], ssems[r]).wait()

    for r in range(PF):
        start_gather(r, r)

    def wave(w, carry):
        for r in range(RING):
            g = w * RING + r
            wait_gather(g, r)
            start_scatter(g, r)
            gp = g + PF
            rp = (r + PF) % RING

            @pl.when(gp < GROUPS)
            def _():
                @pl.when(gp >= RING)
                def _():
                    wait_scatter(rp)
                start_gather(gp, rp)
        return carry
    lax.fori_loop(0, WAVES, wave, 0)

    for r in range(RING):
        wait_scatter(r)


@jax.jit
def _node_to_edge_triple(hv_flat, i1, i2, i3):
    mesh = plsc.VectorSubcoreMesh(core_axis_name="c", subcore_axis_name="s")
    scratch = [
        pltpu.VMEM((BN_W,), jnp.int32),
        pltpu.VMEM((BN_W,), jnp.int32),
        pltpu.VMEM((BN_W,), jnp.int32),
        pltpu.VMEM((GROUPS, GR), jnp.int32),
        pltpu.VMEM((RING, GR, D), jnp.float32),
        pltpu.VMEM_SHARED((B * V, D), jnp.float32),
    ] + [pltpu.SemaphoreType.DMA] * (2 * RING)
    fn = pl.kernel(
        _sc_body,
        mesh=mesh,
        out_type=jax.ShapeDtypeStruct((BN * 3, D), jnp.float32),
        scratch_types=scratch,
        compiler_params=pltpu.CompilerParams(needs_layout_passes=False),
    )
    return fn(hv_flat, i1, i2, i3)


def kernel(hv, v1s_idx, v2s_idx, v3d_idx):
    hv_flat = hv.reshape(B * V, D)
    out = _node_to_edge_triple(
        hv_flat,
        v1s_idx.astype(jnp.int32),
        v2s_idx.astype(jnp.int32),
        v3d_idx.astype(jnp.int32),
    )
    return out.reshape(B, V, V, V, 3 * D)


# speedup vs baseline: 1.0071x; 1.0071x over previous
import functools

import jax
import jax.numpy as jnp
from jax import lax
from jax.experimental import pallas as pl
from jax.experimental.pallas import tpu as pltpu
from jax.experimental.pallas import tpu_sc as plsc

B, V, D = 4, 32, 128
N = V * V * V
NC, NS = 2, 16
NW = NC * NS
BN = B * N
BN_W = BN // NW
ROWS_W = BN_W * 3
GR = 128
GROUPS = ROWS_W // GR
RING = 6
PF = 3
WAVES = GROUPS // RING


def _sc_body(hv_ref, i1_ref, i2_ref, i3_ref, out_ref,
             idx1_v, idx2_v, idx3_v, comb_v, rows_v, table_sh, *sems):
    gsems = sems[:RING]
    ssems = sems[RING:]

    wid = lax.axis_index("s") * NC + lax.axis_index("c")
    b = wid // (N // BN_W)
    n0 = (wid % (N // BN_W)) * BN_W
    b_off = b * V
    row0 = wid * ROWS_W

    @pl.when(lax.axis_index("s") == 0)
    def _():
        pltpu.sync_copy(hv_ref, rows_v.at[0])
        pltpu.sync_copy(rows_v.at[0], table_sh)
    plsc.subcore_barrier()

    pltpu.sync_copy(i1_ref.at[pl.ds(n0, BN_W)], idx1_v)
    pltpu.sync_copy(i2_ref.at[pl.ds(n0, BN_W)], idx2_v)
    pltpu.sync_copy(i3_ref.at[pl.ds(n0, BN_W)], idx3_v)

    lane = lax.iota(jnp.int32, 16)

    def fill(c, carry):
        base = c * 16
        for si, src in enumerate((idx1_v, idx2_v, idx3_v)):
            vals = src[pl.ds(base, 16)] + b_off
            p = (base + lane) * 3 + si
            prow = lax.shift_right_logical(p, 7)
            pcol = lax.bitwise_and(p, 127)
            plsc.store_scatter(comb_v, [prow, pcol], vals)
        return carry
    lax.fori_loop(0, BN_W // 16, fill, 0)

    def _src(r):
        return table_sh

    def start_gather(g, r):
        pltpu.async_copy(_src(r).at[comb_v.at[g]], rows_v.at[r], gsems[r])

    def wait_gather(g, r):
        pltpu.make_async_copy(
            _src(r).at[comb_v.at[g]], rows_v.at[r], gsems[r]).wait()

    def start_scatter(g, r):
        pltpu.async_copy(
            rows_v.at[r], out_ref.at[pl.ds(row0 + g * GR, GR)], ssems[r])

    def wait_scatter(r):
        pltpu.make_async_copy(
            rows_v.at[r], out_ref.at[pl.ds(row0, GR)], ssems[r]).wait()

    for r in range(PF):
        start_gather(r, r)

    def wave(w, carry):
        for r in range(RING):
            g = w * RING + r
            wait_gather(g, r)
            start_scatter(g, r)
            gp = g + PF
            rp = (r + PF) % RING

            @pl.when(gp < GROUPS)
            def _():
                @pl.when(gp >= RING)
                def _():
                    wait_scatter(rp)
                start_gather(gp, rp)
        return carry
    lax.fori_loop(0, WAVES, wave, 0)

    for r in range(RING):
        wait_scatter(r)


@jax.jit
def _node_to_edge_triple(hv_flat, i1, i2, i3):
    mesh = plsc.VectorSubcoreMesh(core_axis_name="c", subcore_axis_name="s")
    scratch = [
        pltpu.VMEM((BN_W,), jnp.int32),
        pltpu.VMEM((BN_W,), jnp.int32),
        pltpu.VMEM((BN_W,), jnp.int32),
        pltpu.VMEM((GROUPS, GR), jnp.int32),
        pltpu.VMEM((RING, GR, D), jnp.float32),
        pltpu.VMEM_SHARED((B * V, D), jnp.float32),
    ] + [pltpu.SemaphoreType.DMA] * (2 * RING)
    fn = pl.kernel(
        _sc_body,
        mesh=mesh,
        out_type=jax.ShapeDtypeStruct((BN * 3, D), jnp.float32),
        scratch_types=scratch,
        compiler_params=pltpu.CompilerParams(needs_layout_passes=False),
    )
    return fn(hv_flat, i1, i2, i3)


def kernel(hv, v1s_idx, v2s_idx, v3d_idx):
    hv_flat = hv.reshape(B * V, D)
    out = _node_to_edge_triple(
        hv_flat,
        v1s_idx.astype(jnp.int32),
        v2s_idx.astype(jnp.int32),
        v3d_idx.astype(jnp.int32),
    )
    return out.reshape(B, V, V, V, 3 * D)


# speedup vs baseline: 3.0564x; 3.0350x over previous
import jax
import jax.numpy as jnp
from jax import lax
from jax.experimental import pallas as pl
from jax.experimental.pallas import tpu as pltpu
from jax.experimental.pallas import tpu_sc as plsc

B, V, D = 4, 32, 128
N = V * V * V
NC, NS = 2, 16
NW = NC * NS
BN = B * N
BN_W = BN // NW
TB = 128
BLOCKS = BN_W // TB
JB = TB // V
GROUPS = BLOCKS * 3
RING = 6
PF = 3
WAVES = GROUPS // RING


def _sc_body(hv_ref, i1_ref, i2_ref, i3_ref, out_ref,
             idx1_v, idx2_v, idx3_v, rows_v, stage_v, table_sh, *sems):
    gsems = sems[:RING]
    ssems = sems[RING:]

    wid = lax.axis_index("s") * NC + lax.axis_index("c")
    b = wid // (N // BN_W)
    n0 = (wid % (N // BN_W)) * BN_W
    i0 = n0 // (V * V)
    row0 = (b * V + i0) * V
    b_off = b * V

    @pl.when(lax.axis_index("s") == 0)
    def _():
        pltpu.sync_copy(hv_ref, stage_v)
        pltpu.sync_copy(stage_v, table_sh)
    plsc.subcore_barrier()

    pltpu.sync_copy(i1_ref.at[pl.ds(n0, BN_W)], idx1_v)
    pltpu.sync_copy(i2_ref.at[pl.ds(n0, BN_W)], idx2_v)
    pltpu.sync_copy(i3_ref.at[pl.ds(n0, BN_W)], idx3_v)

    def bias(c, carry):
        base = c * 16
        for src in (idx1_v, idx2_v, idx3_v):
            src[pl.ds(base, 16)] = src[pl.ds(base, 16)] + b_off
        return carry
    lax.fori_loop(0, BN_W // 16, bias, 0)

    idxs = (idx1_v, idx2_v, idx3_v)

    def start_gather(g, r, s):
        m = g // 3
        for jj in range(JB):
            pltpu.async_copy(
                table_sh.at[idxs[s].at[pl.ds(m * TB + jj * V, V)]],
                rows_v.at[r, jj], gsems[r])

    def wait_gather(g, r, s):
        m = g // 3
        for jj in range(JB):
            pltpu.make_async_copy(
                table_sh.at[idxs[s].at[pl.ds(m * TB + jj * V, V)]],
                rows_v.at[r, jj], gsems[r]).wait()

    def _dst(g, s):
        m = g // 3
        return out_ref.at[pl.ds(row0 + m * JB, JB), slice(None),
                          pl.ds(s * D, D)]

    def start_scatter(g, r, s):
        pltpu.async_copy(rows_v.at[r], _dst(g, s), ssems[r])

    def wait_scatter(g, r, s):
        pltpu.make_async_copy(rows_v.at[r], _dst(g, s), ssems[r]).wait()

    for r in range(PF):
        start_gather(r, r, r % 3)

    def wave(w, carry):
        for r in range(RING):
            g = w * RING + r
            wait_gather(g, r, r % 3)
            start_scatter(g, r, r % 3)
            gp = g + PF
            rp = (r + PF) % RING

            @pl.when(gp < GROUPS)
            def _():
                @pl.when(gp >= RING)
                def _():
                    wait_scatter(gp - RING, rp, rp % 3)
                start_gather(gp, rp, rp % 3)
        return carry
    lax.fori_loop(0, WAVES, wave, 0)

    for r in range(RING):
        wait_scatter(GROUPS - RING + r, r, r % 3)


@jax.jit
def _node_to_edge_triple(hv_flat, i1, i2, i3):
    mesh = plsc.VectorSubcoreMesh(core_axis_name="c", subcore_axis_name="s")
    scratch = [
        pltpu.VMEM((BN_W,), jnp.int32),
        pltpu.VMEM((BN_W,), jnp.int32),
        pltpu.VMEM((BN_W,), jnp.int32),
        pltpu.VMEM((RING, JB, V, D), jnp.float32),
        pltpu.VMEM((B * V, D), jnp.float32),
        pltpu.VMEM_SHARED((B * V, D), jnp.float32),
    ] + [pltpu.SemaphoreType.DMA] * (2 * RING)
    fn = pl.kernel(
        _sc_body,
        mesh=mesh,
        out_type=jax.ShapeDtypeStruct((B * V * V, V, 3 * D), jnp.float32),
        scratch_types=scratch,
        compiler_params=pltpu.CompilerParams(needs_layout_passes=False),
    )
    return fn(hv_flat, i1, i2, i3)


def kernel(hv, v1s_idx, v2s_idx, v3d_idx):
    hv_flat = hv.reshape(B * V, D)
    out = _node_to_edge_triple(
        hv_flat,
        v1s_idx.astype(jnp.int32),
        v2s_idx.astype(jnp.int32),
        v3d_idx.astype(jnp.int32),
    )
    return out.reshape(B, V, V, V, 3 * D)
